# baseline (speedup 1.0000x reference)
import functools

import numpy as np
import jax
import jax.numpy as jnp
from jax import lax
from jax.experimental import pallas as pl
from jax.experimental.pallas import tpu as pltpu
from jax.experimental.pallas import tpu_sc as plsc

N_NODES = 10000
N_EDGES = 160000
IN_CH = 256
OUT_CH = 256
N_FILTERS = 256
N_GAUSS = 64
CUTOFF = 10.0
HALF = N_FILTERS // 2

_centers = np.linspace(0.0, CUTOFF, N_GAUSS).astype(np.float32)
_SPACING = np.float32(_centers[1] - _centers[0])
_coeff = np.float32(-0.5 / (_SPACING * _SPACING))
_LOG2 = np.float32(np.log(2.0))
_LN2 = np.float32(np.log(2.0))

NC, NS = 2, 16
EPW = N_EDGES // (NC * NS)
K2 = 80
NCHUNK_M = N_EDGES // K2
T_M = (NCHUNK_M + NS - 1) // NS


def _hw_body(h_ref, win_ref, out_ref):
    hblk = h_ref[...]
    w = win_ref[...]
    out_ref[0] = jnp.dot(hblk, w[:, :HALF], preferred_element_type=jnp.float32)
    out_ref[1] = jnp.dot(hblk, w[:, HALF:], preferred_element_type=jnp.float32)


def _hw_call(h, Win):
    BN = 1000
    return pl.pallas_call(
        _hw_body,
        grid=(N_NODES // BN,),
        in_specs=[
            pl.BlockSpec((BN, IN_CH), lambda i: (i, 0)),
            pl.BlockSpec((IN_CH, N_FILTERS), lambda i: (0, 0)),
        ],
        out_specs=pl.BlockSpec((2, BN, HALF), lambda i: (0, i, 0)),
        out_shape=jax.ShapeDtypeStruct((2, N_NODES, HALF), jnp.float32),
    )(h, Win)


def _d2_body(coordf, row, col, d2out, idxr_v, idxc_v, d2_v, tab_v):
    c = lax.axis_index("c")
    s = lax.axis_index("s")
    w = s * NC + c
    base = w * EPW

    pltpu.sync_copy(coordf, tab_v)
    pltpu.sync_copy(row.at[pl.ds(base, EPW)], idxr_v.at[pl.ds(0, EPW)])
    pltpu.sync_copy(col.at[pl.ds(base, EPW)], idxc_v.at[pl.ds(0, EPW)])

    def grp(g, carry):
        sl = pl.ds(g * 16, 16)
        br = jnp.minimum(jnp.maximum(idxr_v[sl], 0), N_NODES - 1) * 4
        bc = jnp.minimum(jnp.maximum(idxc_v[sl], 0), N_NODES - 1) * 4
        dx = plsc.load_gather(tab_v, [br]) - plsc.load_gather(tab_v, [bc])
        dy = (plsc.load_gather(tab_v, [br + 1])
              - plsc.load_gather(tab_v, [bc + 1]))
        dz = (plsc.load_gather(tab_v, [br + 2])
              - plsc.load_gather(tab_v, [bc + 2]))
        d2_v[sl] = dx * dx + dy * dy + dz * dz
        return carry

    lax.fori_loop(0, (EPW + 15) // 16, grp, 0)
    pltpu.sync_copy(d2_v.at[pl.ds(0, EPW)], d2out.at[pl.ds(base, EPW)])


_PAD = 16 * ((EPW + 15) // 16)

_d2_call = pl.kernel(
    _d2_body,
    out_type=jax.ShapeDtypeStruct((N_EDGES,), jnp.float32),
    mesh=plsc.VectorSubcoreMesh(core_axis_name="c", subcore_axis_name="s"),
    scratch_types=[
        pltpu.VMEM((_PAD,), jnp.int32),
        pltpu.VMEM((_PAD,), jnp.int32),
        pltpu.VMEM((_PAD,), jnp.float32),
        pltpu.VMEM((4 * N_NODES,), jnp.float32),
    ],
    compiler_params=pltpu.CompilerParams(needs_layout_passes=False),
)


def _filter_body(d2_ref, mask_ref, w1_ref, b1_ref, w2_ref, b2_ref, out_ref):
    d2 = d2_ref[...]
    d = jnp.minimum(jnp.sqrt(d2), CUTOFF)
    centers = _SPACING * lax.broadcasted_iota(
        jnp.int32, (1, N_GAUSS), 1).astype(jnp.float32)
    delta = d - centers
    rbf = jnp.exp(_coeff * (delta * delta))
    t = jnp.dot(rbf, w1_ref[...], preferred_element_type=jnp.float32)
    t = t + b1_ref[...][None, :]
    ssp = jnp.log(1.0 + jnp.exp(t)) - _LOG2
    wf = jnp.dot(ssp, w2_ref[...], preferred_element_type=jnp.float32)
    wf = (wf + b2_ref[...][None, :]) * mask_ref[...]
    out_ref[0] = wf[:, :HALF]
    out_ref[1] = wf[:, HALF:]


def _filter_call(d2, mask, W1, b1, W2, b2):
    BE = 800
    return pl.pallas_call(
        _filter_body,
        grid=(N_EDGES // BE,),
        in_specs=[
            pl.BlockSpec((BE, 1), lambda i: (i, 0)),
            pl.BlockSpec((BE, 1), lambda i: (i, 0)),
            pl.BlockSpec((N_GAUSS, N_FILTERS), lambda i: (0, 0)),
            pl.BlockSpec((N_FILTERS,), lambda i: (0,)),
            pl.BlockSpec((N_FILTERS, N_FILTERS), lambda i: (0, 0)),
            pl.BlockSpec((N_FILTERS,), lambda i: (0,)),
        ],
        out_specs=pl.BlockSpec((2, BE, HALF), lambda i: (0, i, 0)),
        out_shape=jax.ShapeDtypeStruct((2, N_EDGES, HALF), jnp.float32),
    )(d2, mask, W1, b1, W2, b2)


def _msg_body(hw, wmat, rc, agg,
              ic0, ic1, ir0, ir1, pc0, pc1, a0, a1, w0, w1, z_v,
              agg_sh, semi0, semi1, semo0, semo1, semx0, semx1):
    c = lax.axis_index("c")
    s = lax.axis_index("s")
    av = (a0, a1)
    wv = (w0, w1)
    icv = (ic0, ic1)
    irv = (ir0, ir1)
    pcv = (pc0, pc1)
    semi = (semi0, semi1)
    semo = (semo0, semo1)
    semx = (semx0, semx1)

    def issue(t, b, first):
        j = s + NS * t

        @pl.when(j < NCHUNK_M)
        def _():
            if not first:
                pltpu.make_async_copy(
                    av[b], agg_sh.at[irv[b]], semo[b]).wait()
                pltpu.make_async_copy(
                    rc.at[pl.ds(j * K2, K2)], pcv[b], semx[b]).wait()

            base = j * K2

            def unp(g, cc):
                sl = pl.ds(g * 16, 16)
                pk = pcv[b][sl]
                irv[b][sl] = lax.bitwise_and(pk, 0xFFFF)
                icv[b][sl] = lax.shift_right_logical(pk, 16)
                return cc

            lax.fori_loop(0, K2 // 16, unp, 0)
            pltpu.async_copy(hw.at[c].at[icv[b]], av[b], semi[b])
            pltpu.async_copy(wmat.at[c].at[pl.ds(base, K2)], wv[b], semi[b])

    def compute(t, b):
        j = s + NS * t

        @pl.when(j < NCHUNK_M)
        def _():
            base = j * K2
            pltpu.make_async_copy(hw.at[c].at[icv[b]], av[b], semi[b]).wait()
            pltpu.make_async_copy(
                wmat.at[c].at[pl.ds(base, K2)], wv[b], semi[b]).wait()

            j2 = j + 2 * NS

            @pl.when(j2 < NCHUNK_M)
            def _():
                pltpu.async_copy(rc.at[pl.ds(j2 * K2, K2)], pcv[b], semx[b])

            def mrow(i, cc):
                for rr in range(4):
                    r = i * 4 + rr
                    for kk in range(HALF // 16):
                        sl = pl.ds(kk * 16, 16)
                        av[b][r, sl] = av[b][r, sl] * wv[b][r, sl]
                return cc

            lax.fori_loop(0, K2 // 4, mrow, 0)
            pltpu.async_copy(av[b], agg_sh.at[irv[b]], semo[b], add=True)

    @pl.when(s < NCHUNK_M)
    def _():
        pltpu.sync_copy(rc.at[pl.ds(s * K2, K2)], pcv[0])

    @pl.when(s + NS < NCHUNK_M)
    def _():
        pltpu.sync_copy(rc.at[pl.ds((s + NS) * K2, K2)], pcv[1])

    issue(0, 0, True)
    issue(1, 1, True)

    def zstore(i, carry):
        r = i // (HALF // 16)
        kk = (i % (HALF // 16)) * 16
        z_v[r, pl.ds(kk, 16)] = jnp.zeros((16,), jnp.float32)
        return carry

    lax.fori_loop(0, 16 * (HALF // 16), zstore, 0)

    base0 = s * 624
    nrows = jnp.where(s == NS - 1, 640, 624)

    def zcopy(i, carry):
        pltpu.sync_copy(z_v, agg_sh.at[pl.ds(base0 + i * 16, 16)])
        return carry

    lax.fori_loop(0, nrows // 16, zcopy, 0)
    plsc.subcore_barrier()

    compute(0, 0)

    def body(tt, carry):
        t0 = 2 * tt
        issue(t0 + 2, 0, False)
        compute(t0 + 1, 1)
        issue(t0 + 3, 1, False)
        compute(t0 + 2, 0)
        return carry

    lax.fori_loop(0, (T_M + 1) // 2, body, 0)

    pltpu.make_async_copy(av[0], agg_sh.at[irv[0]], semo[0]).wait()
    pltpu.make_async_copy(av[1], agg_sh.at[irv[1]], semo[1]).wait()

    plsc.subcore_barrier()

    @pl.when(s == NS - 1)
    def _():
        pltpu.sync_copy(agg_sh.at[pl.ds(base0, 640)],
                        agg.at[c].at[pl.ds(base0, 640)])

    @pl.when(s < NS - 1)
    def _():
        pltpu.sync_copy(agg_sh.at[pl.ds(base0, 624)],
                        agg.at[c].at[pl.ds(base0, 624)])


_msg_call = pl.kernel(
    _msg_body,
    out_type=jax.ShapeDtypeStruct((2, N_NODES, HALF), jnp.float32),
    mesh=plsc.VectorSubcoreMesh(core_axis_name="c", subcore_axis_name="s"),
    scratch_types=[
        pltpu.VMEM((K2,), jnp.int32),
        pltpu.VMEM((K2,), jnp.int32),
        pltpu.VMEM((K2,), jnp.int32),
        pltpu.VMEM((K2,), jnp.int32),
        pltpu.VMEM((K2,), jnp.int32),
        pltpu.VMEM((K2,), jnp.int32),
        pltpu.VMEM((K2, HALF), jnp.float32),
        pltpu.VMEM((K2, HALF), jnp.float32),
        pltpu.VMEM((K2, HALF), jnp.float32),
        pltpu.VMEM((K2, HALF), jnp.float32),
        pltpu.VMEM((16, HALF), jnp.float32),
        pltpu.VMEM_SHARED((N_NODES, HALF), jnp.float32),
        pltpu.SemaphoreType.DMA,
        pltpu.SemaphoreType.DMA,
        pltpu.SemaphoreType.DMA,
        pltpu.SemaphoreType.DMA,
        pltpu.SemaphoreType.DMA,
        pltpu.SemaphoreType.DMA,
    ],
    compiler_params=pltpu.CompilerParams(needs_layout_passes=False),
)


def _out_body(agg_ref, wout_ref, bout_ref, out_ref):
    w = wout_ref[...]
    acc = jnp.dot(agg_ref[0], w[:HALF, :], preferred_element_type=jnp.float32)
    acc = acc + jnp.dot(agg_ref[1], w[HALF:, :],
                        preferred_element_type=jnp.float32)
    out_ref[...] = acc + bout_ref[...][None, :]


def _out_call(agg, Wout, bout):
    BN = 1000
    return pl.pallas_call(
        _out_body,
        grid=(N_NODES // BN,),
        in_specs=[
            pl.BlockSpec((2, BN, HALF), lambda i: (0, i, 0)),
            pl.BlockSpec((N_FILTERS, OUT_CH), lambda i: (0, 0)),
            pl.BlockSpec((OUT_CH,), lambda i: (0,)),
        ],
        out_specs=pl.BlockSpec((BN, OUT_CH), lambda i: (i, 0)),
        out_shape=jax.ShapeDtypeStruct((N_NODES, OUT_CH), jnp.float32),
    )(agg, Wout, bout)


def kernel(h, coord, edge_index, edge_mask, W1, b1, W2, b2, Win, Wout, bout):
    row = edge_index[0].astype(jnp.int32)
    col = edge_index[1].astype(jnp.int32)
    coordf = jnp.concatenate(
        [coord, jnp.zeros((N_NODES, 1), coord.dtype)], axis=1).reshape(-1)

    hw = _hw_call(h, Win)
    d2 = _d2_call(coordf, row, col)
    wmat = _filter_call(d2.reshape(N_EDGES, 1), edge_mask, W1, b1, W2, b2)
    rc = jnp.bitwise_or(row, jnp.left_shift(col, 16))
    agg = _msg_call(hw, wmat, rc)
    return _out_call(agg, Wout, bout)


# speedup vs baseline: 1.1147x; 1.1147x over previous
import functools

import numpy as np
import jax
import jax.numpy as jnp
from jax import lax
from jax.experimental import pallas as pl
from jax.experimental.pallas import tpu as pltpu
from jax.experimental.pallas import tpu_sc as plsc

N_NODES = 10000
N_EDGES = 160000
IN_CH = 256
OUT_CH = 256
N_FILTERS = 256
N_GAUSS = 64
CUTOFF = 10.0
HALF = N_FILTERS // 2

_centers = np.linspace(0.0, CUTOFF, N_GAUSS).astype(np.float32)
_SPACING = np.float32(_centers[1] - _centers[0])
_coeff = np.float32(-0.5 / (_SPACING * _SPACING))
_LOG2 = np.float32(np.log(2.0))
_LN2 = np.float32(np.log(2.0))

NC, NS = 2, 16
EPW = N_EDGES // (NC * NS)
K2 = 80
NCHUNK_M = N_EDGES // K2
T_M = (NCHUNK_M + NS - 1) // NS


def _hw_body(h_ref, win_ref, out_ref):
    hblk = h_ref[...]
    w = win_ref[...]
    out_ref[0] = jnp.dot(hblk, w[:, :HALF], preferred_element_type=jnp.float32)
    out_ref[1] = jnp.dot(hblk, w[:, HALF:], preferred_element_type=jnp.float32)


def _hw_call(h, Win):
    BN = 1000
    return pl.pallas_call(
        _hw_body,
        grid=(N_NODES // BN,),
        in_specs=[
            pl.BlockSpec((BN, IN_CH), lambda i: (i, 0)),
            pl.BlockSpec((IN_CH, N_FILTERS), lambda i: (0, 0)),
        ],
        out_specs=pl.BlockSpec((2, BN, HALF), lambda i: (0, i, 0)),
        out_shape=jax.ShapeDtypeStruct((2, N_NODES, HALF), jnp.float32),
    )(h, Win)


def _d2_body(coordf, row, col, d2out, idxr_v, idxc_v, d2_v, tab_v):
    c = lax.axis_index("c")
    s = lax.axis_index("s")
    w = s * NC + c
    base = w * EPW

    pltpu.sync_copy(coordf, tab_v)
    pltpu.sync_copy(row.at[pl.ds(base, EPW)], idxr_v.at[pl.ds(0, EPW)])
    pltpu.sync_copy(col.at[pl.ds(base, EPW)], idxc_v.at[pl.ds(0, EPW)])

    def grp(g, carry):
        sl = pl.ds(g * 16, 16)
        br = jnp.minimum(jnp.maximum(idxr_v[sl], 0), N_NODES - 1) * 4
        bc = jnp.minimum(jnp.maximum(idxc_v[sl], 0), N_NODES - 1) * 4
        dx = plsc.load_gather(tab_v, [br]) - plsc.load_gather(tab_v, [bc])
        dy = (plsc.load_gather(tab_v, [br + 1])
              - plsc.load_gather(tab_v, [bc + 1]))
        dz = (plsc.load_gather(tab_v, [br + 2])
              - plsc.load_gather(tab_v, [bc + 2]))
        d2_v[sl] = dx * dx + dy * dy + dz * dz
        return carry

    lax.fori_loop(0, (EPW + 15) // 16, grp, 0)
    pltpu.sync_copy(d2_v.at[pl.ds(0, EPW)], d2out.at[pl.ds(base, EPW)])


_PAD = 16 * ((EPW + 15) // 16)

_d2_call = pl.kernel(
    _d2_body,
    out_type=jax.ShapeDtypeStruct((N_EDGES,), jnp.float32),
    mesh=plsc.VectorSubcoreMesh(core_axis_name="c", subcore_axis_name="s"),
    scratch_types=[
        pltpu.VMEM((_PAD,), jnp.int32),
        pltpu.VMEM((_PAD,), jnp.int32),
        pltpu.VMEM((_PAD,), jnp.float32),
        pltpu.VMEM((4 * N_NODES,), jnp.float32),
    ],
    compiler_params=pltpu.CompilerParams(needs_layout_passes=False),
)


def _filter_body(d2_ref, mask_ref, w1_ref, b1_ref, w2_ref, b2_ref, out_ref):
    d2 = d2_ref[...]
    d = jnp.minimum(jnp.sqrt(d2), CUTOFF)
    centers = _SPACING * lax.broadcasted_iota(
        jnp.int32, (1, N_GAUSS), 1).astype(jnp.float32)
    delta = d - centers
    rbf = jnp.exp(_coeff * (delta * delta))
    t = jnp.dot(rbf, w1_ref[...], preferred_element_type=jnp.float32)
    t = t + b1_ref[...][None, :]
    ssp = jnp.log(1.0 + jnp.exp(t)) - _LOG2
    wf = jnp.dot(ssp, w2_ref[...], preferred_element_type=jnp.float32)
    wf = (wf + b2_ref[...][None, :]) * mask_ref[...]
    out_ref[0] = wf[:, :HALF]
    out_ref[1] = wf[:, HALF:]


def _filter_call(d2, mask, W1, b1, W2, b2):
    ne = d2.shape[0]
    BE = 800
    return pl.pallas_call(
        _filter_body,
        grid=(ne // BE,),
        in_specs=[
            pl.BlockSpec((BE, 1), lambda i: (i, 0)),
            pl.BlockSpec((BE, 1), lambda i: (i, 0)),
            pl.BlockSpec((N_GAUSS, N_FILTERS), lambda i: (0, 0)),
            pl.BlockSpec((N_FILTERS,), lambda i: (0,)),
            pl.BlockSpec((N_FILTERS, N_FILTERS), lambda i: (0, 0)),
            pl.BlockSpec((N_FILTERS,), lambda i: (0,)),
        ],
        out_specs=pl.BlockSpec((2, BE, HALF), lambda i: (0, i, 0)),
        out_shape=jax.ShapeDtypeStruct((2, ne, HALF), jnp.float32),
    )(d2, mask, W1, b1, W2, b2)


def _make_msg(nchunk):
  tm = (nchunk + NS - 1) // NS

  def _msg_body(hw, wmat, rc, agg,
                ic0, ic1, ir0, ir1, pc0, pc1, a0, a1, w0, w1, z_v,
                agg_sh, semi0, semi1, semo0, semo1, semx0, semx1):
      c = lax.axis_index("c")
      s = lax.axis_index("s")
      av = (a0, a1)
      wv = (w0, w1)
      icv = (ic0, ic1)
      irv = (ir0, ir1)
      pcv = (pc0, pc1)
      semi = (semi0, semi1)
      semo = (semo0, semo1)
      semx = (semx0, semx1)

      def issue(t, b, first):
          j = s + NS * t

          @pl.when(j < nchunk)
          def _():
              if not first:
                  pltpu.make_async_copy(
                      av[b], agg_sh.at[irv[b]], semo[b]).wait()
                  pltpu.make_async_copy(
                      rc.at[pl.ds(j * K2, K2)], pcv[b], semx[b]).wait()

              base = j * K2

              def unp(g, cc):
                  sl = pl.ds(g * 16, 16)
                  pk = pcv[b][sl]
                  irv[b][sl] = lax.bitwise_and(pk, 0xFFFF)
                  icv[b][sl] = lax.shift_right_logical(pk, 16)
                  return cc

              lax.fori_loop(0, K2 // 16, unp, 0)
              pltpu.async_copy(hw.at[c].at[icv[b]], av[b], semi[b])
              pltpu.async_copy(wmat.at[c].at[pl.ds(base, K2)], wv[b], semi[b])

      def compute(t, b):
          j = s + NS * t

          @pl.when(j < nchunk)
          def _():
              base = j * K2
              pltpu.make_async_copy(hw.at[c].at[icv[b]], av[b], semi[b]).wait()
              pltpu.make_async_copy(
                  wmat.at[c].at[pl.ds(base, K2)], wv[b], semi[b]).wait()

              j2 = j + 2 * NS

              @pl.when(j2 < nchunk)
              def _():
                  pltpu.async_copy(rc.at[pl.ds(j2 * K2, K2)], pcv[b], semx[b])

              def mrow(i, cc):
                  for rr in range(2):
                      r = i * 2 + rr
                      for kk in range(HALF // 16):
                          sl = pl.ds(kk * 16, 16)
                          av[b][r, sl] = av[b][r, sl] * wv[b][r, sl]
                  return cc

              lax.fori_loop(0, K2 // 2, mrow, 0)
              pltpu.async_copy(av[b], agg_sh.at[irv[b]], semo[b], add=True)

      @pl.when(s < nchunk)
      def _():
          pltpu.sync_copy(rc.at[pl.ds(s * K2, K2)], pcv[0])

      @pl.when(s + NS < nchunk)
      def _():
          pltpu.sync_copy(rc.at[pl.ds((s + NS) * K2, K2)], pcv[1])

      issue(0, 0, True)
      issue(1, 1, True)

      def zstore(i, carry):
          r = i // (HALF // 16)
          kk = (i % (HALF // 16)) * 16
          z_v[r, pl.ds(kk, 16)] = jnp.zeros((16,), jnp.float32)
          return carry

      lax.fori_loop(0, 16 * (HALF // 16), zstore, 0)

      base0 = s * 624
      nrows = jnp.where(s == NS - 1, 640, 624)

      def zcopy(i, carry):
          pltpu.sync_copy(z_v, agg_sh.at[pl.ds(base0 + i * 16, 16)])
          return carry

      lax.fori_loop(0, nrows // 16, zcopy, 0)
      plsc.subcore_barrier()

      compute(0, 0)

      def body(tt, carry):
          t0 = 2 * tt
          issue(t0 + 2, 0, False)
          compute(t0 + 1, 1)
          issue(t0 + 3, 1, False)
          compute(t0 + 2, 0)
          return carry

      lax.fori_loop(0, (tm + 1) // 2, body, 0)

      pltpu.make_async_copy(av[0], agg_sh.at[irv[0]], semo[0]).wait()
      pltpu.make_async_copy(av[1], agg_sh.at[irv[1]], semo[1]).wait()

      plsc.subcore_barrier()

      @pl.when(s == NS - 1)
      def _():
          pltpu.sync_copy(agg_sh.at[pl.ds(base0, 640)],
                          agg.at[c].at[pl.ds(base0, 640)])

      @pl.when(s < NS - 1)
      def _():
          pltpu.sync_copy(agg_sh.at[pl.ds(base0, 624)],
                          agg.at[c].at[pl.ds(base0, 624)])

  return pl.kernel(
      _msg_body,
      out_type=jax.ShapeDtypeStruct((2, N_NODES, HALF), jnp.float32),
      mesh=plsc.VectorSubcoreMesh(core_axis_name="c", subcore_axis_name="s"),
      scratch_types=[
          pltpu.VMEM((K2,), jnp.int32),
          pltpu.VMEM((K2,), jnp.int32),
          pltpu.VMEM((K2,), jnp.int32),
          pltpu.VMEM((K2,), jnp.int32),
          pltpu.VMEM((K2,), jnp.int32),
          pltpu.VMEM((K2,), jnp.int32),
          pltpu.VMEM((K2, HALF), jnp.float32),
          pltpu.VMEM((K2, HALF), jnp.float32),
          pltpu.VMEM((K2, HALF), jnp.float32),
          pltpu.VMEM((K2, HALF), jnp.float32),
          pltpu.VMEM((16, HALF), jnp.float32),
          pltpu.VMEM_SHARED((N_NODES, HALF), jnp.float32),
          pltpu.SemaphoreType.DMA,
          pltpu.SemaphoreType.DMA,
          pltpu.SemaphoreType.DMA,
          pltpu.SemaphoreType.DMA,
          pltpu.SemaphoreType.DMA,
          pltpu.SemaphoreType.DMA,
      ],
      compiler_params=pltpu.CompilerParams(needs_layout_passes=False),
  )


_msg_half = _make_msg(N_EDGES // (2 * K2))


def _out_body(agga_ref, aggb_ref, wout_ref, bout_ref, out_ref):
    w = wout_ref[...]
    a0 = agga_ref[0] + aggb_ref[0]
    a1 = agga_ref[1] + aggb_ref[1]
    acc = jnp.dot(a0, w[:HALF, :], preferred_element_type=jnp.float32)
    acc = acc + jnp.dot(a1, w[HALF:, :], preferred_element_type=jnp.float32)
    out_ref[...] = acc + bout_ref[...][None, :]


def _out_call(agga, aggb, Wout, bout):
    BN = 1000
    return pl.pallas_call(
        _out_body,
        grid=(N_NODES // BN,),
        in_specs=[
            pl.BlockSpec((2, BN, HALF), lambda i: (0, i, 0)),
            pl.BlockSpec((2, BN, HALF), lambda i: (0, i, 0)),
            pl.BlockSpec((N_FILTERS, OUT_CH), lambda i: (0, 0)),
            pl.BlockSpec((OUT_CH,), lambda i: (0,)),
        ],
        out_specs=pl.BlockSpec((BN, OUT_CH), lambda i: (i, 0)),
        out_shape=jax.ShapeDtypeStruct((N_NODES, OUT_CH), jnp.float32),
    )(agga, aggb, Wout, bout)


def kernel(h, coord, edge_index, edge_mask, W1, b1, W2, b2, Win, Wout, bout):
    row = edge_index[0].astype(jnp.int32)
    col = edge_index[1].astype(jnp.int32)
    coordf = jnp.concatenate(
        [coord, jnp.zeros((N_NODES, 1), coord.dtype)], axis=1).reshape(-1)

    hw = _hw_call(h, Win)
    d2 = _d2_call(coordf, row, col)
    rc = jnp.bitwise_or(row, jnp.left_shift(col, 16))
    eh = N_EDGES // 2
    wma = _filter_call(d2[:eh].reshape(eh, 1), edge_mask[:eh], W1, b1, W2, b2)
    agga = _msg_half(hw, wma, rc[:eh])
    wmb = _filter_call(d2[eh:].reshape(eh, 1), edge_mask[eh:], W1, b1, W2, b2)
    aggb = _msg_half(hw, wmb, rc[eh:])
    return _out_call(agga, aggb, Wout, bout)


# speedup vs baseline: 1.1735x; 1.0527x over previous
import functools

import numpy as np
import jax
import jax.numpy as jnp
from jax import lax
from jax.experimental import pallas as pl
from jax.experimental.pallas import tpu as pltpu
from jax.experimental.pallas import tpu_sc as plsc

N_NODES = 10000
N_EDGES = 160000
IN_CH = 256
OUT_CH = 256
N_FILTERS = 256
N_GAUSS = 64
CUTOFF = 10.0
HALF = N_FILTERS // 2

_centers = np.linspace(0.0, CUTOFF, N_GAUSS).astype(np.float32)
_SPACING = np.float32(_centers[1] - _centers[0])
_coeff = np.float32(-0.5 / (_SPACING * _SPACING))
_LOG2 = np.float32(np.log(2.0))
_LN2 = np.float32(np.log(2.0))

NC, NS = 2, 16
EPW = N_EDGES // (NC * NS)
K2 = 80
NCHUNK_M = N_EDGES // K2
T_M = (NCHUNK_M + NS - 1) // NS


def _hw_body(h_ref, win_ref, out_ref):
    hblk = h_ref[...]
    w = win_ref[...]
    out_ref[0] = jnp.dot(hblk, w[:, :HALF], preferred_element_type=jnp.float32)
    out_ref[1] = jnp.dot(hblk, w[:, HALF:], preferred_element_type=jnp.float32)


def _hw_call(h, Win):
    BN = 1000
    return pl.pallas_call(
        _hw_body,
        grid=(N_NODES // BN,),
        in_specs=[
            pl.BlockSpec((BN, IN_CH), lambda i: (i, 0)),
            pl.BlockSpec((IN_CH, N_FILTERS), lambda i: (0, 0)),
        ],
        out_specs=pl.BlockSpec((2, BN, HALF), lambda i: (0, i, 0)),
        out_shape=jax.ShapeDtypeStruct((2, N_NODES, HALF), jnp.float32),
    )(h, Win)


def _d2_body(coordf, row, col, d2out, idxr_v, idxc_v, d2_v, tab_v):
    c = lax.axis_index("c")
    s = lax.axis_index("s")
    w = s * NC + c
    base = w * EPW

    pltpu.sync_copy(coordf, tab_v)
    pltpu.sync_copy(row.at[pl.ds(base, EPW)], idxr_v.at[pl.ds(0, EPW)])
    pltpu.sync_copy(col.at[pl.ds(base, EPW)], idxc_v.at[pl.ds(0, EPW)])

    def grp(g, carry):
        sl = pl.ds(g * 16, 16)
        br = jnp.minimum(jnp.maximum(idxr_v[sl], 0), N_NODES - 1) * 4
        bc = jnp.minimum(jnp.maximum(idxc_v[sl], 0), N_NODES - 1) * 4
        dx = plsc.load_gather(tab_v, [br]) - plsc.load_gather(tab_v, [bc])
        dy = (plsc.load_gather(tab_v, [br + 1])
              - plsc.load_gather(tab_v, [bc + 1]))
        dz = (plsc.load_gather(tab_v, [br + 2])
              - plsc.load_gather(tab_v, [bc + 2]))
        d2_v[sl] = dx * dx + dy * dy + dz * dz
        return carry

    lax.fori_loop(0, (EPW + 15) // 16, grp, 0)
    pltpu.sync_copy(d2_v.at[pl.ds(0, EPW)], d2out.at[pl.ds(base, EPW)])


_PAD = 16 * ((EPW + 15) // 16)

_d2_call = pl.kernel(
    _d2_body,
    out_type=jax.ShapeDtypeStruct((N_EDGES,), jnp.float32),
    mesh=plsc.VectorSubcoreMesh(core_axis_name="c", subcore_axis_name="s"),
    scratch_types=[
        pltpu.VMEM((_PAD,), jnp.int32),
        pltpu.VMEM((_PAD,), jnp.int32),
        pltpu.VMEM((_PAD,), jnp.float32),
        pltpu.VMEM((4 * N_NODES,), jnp.float32),
    ],
    compiler_params=pltpu.CompilerParams(needs_layout_passes=False),
)


def _filter_body(d2_ref, mask_ref, w1_ref, b1_ref, w2_ref, b2_ref, out_ref):
    d2 = d2_ref[...]
    d = jnp.minimum(jnp.sqrt(d2), CUTOFF)
    centers = _SPACING * lax.broadcasted_iota(
        jnp.int32, (1, N_GAUSS), 1).astype(jnp.float32)
    delta = d - centers
    rbf = jnp.exp(_coeff * (delta * delta))
    t = jnp.dot(rbf, w1_ref[...], preferred_element_type=jnp.float32)
    t = t + b1_ref[...][None, :]
    ssp = jnp.log(1.0 + jnp.exp(t)) - _LOG2
    wf = jnp.dot(ssp, w2_ref[...], preferred_element_type=jnp.float32)
    wf = (wf + b2_ref[...][None, :]) * mask_ref[...]
    out_ref[0] = wf[:, :HALF]
    out_ref[1] = wf[:, HALF:]


def _filter_call(d2, mask, W1, b1, W2, b2):
    ne = d2.shape[0]
    BE = 800
    return pl.pallas_call(
        _filter_body,
        grid=(ne // BE,),
        in_specs=[
            pl.BlockSpec((BE, 1), lambda i: (i, 0)),
            pl.BlockSpec((BE, 1), lambda i: (i, 0)),
            pl.BlockSpec((N_GAUSS, N_FILTERS), lambda i: (0, 0)),
            pl.BlockSpec((N_FILTERS,), lambda i: (0,)),
            pl.BlockSpec((N_FILTERS, N_FILTERS), lambda i: (0, 0)),
            pl.BlockSpec((N_FILTERS,), lambda i: (0,)),
        ],
        out_specs=pl.BlockSpec((2, BE, HALF), lambda i: (0, i, 0)),
        out_shape=jax.ShapeDtypeStruct((2, ne, HALF), jnp.float32),
    )(d2, mask, W1, b1, W2, b2)


def _make_msg(nchunk):
  tm = (nchunk + NS - 1) // NS

  def _msg_body(hw, wmat, rc, agg,
                ic0, ic1, ir0, ir1, pc0, pc1, a0, a1, w0, w1, z_v,
                agg_sh, semi0, semi1, semo0, semo1, semx0, semx1):
      c = lax.axis_index("c")
      s = lax.axis_index("s")
      av = (a0, a1)
      wv = (w0, w1)
      icv = (ic0, ic1)
      irv = (ir0, ir1)
      pcv = (pc0, pc1)
      semi = (semi0, semi1)
      semo = (semo0, semo1)
      semx = (semx0, semx1)

      def issue(t, b, first):
          j = s + NS * t

          @pl.when(j < nchunk)
          def _():
              if not first:
                  pltpu.make_async_copy(
                      av[b], agg_sh.at[irv[b]], semo[b]).wait()
                  pltpu.make_async_copy(
                      rc.at[pl.ds(j * K2, K2)], pcv[b], semx[b]).wait()

              base = j * K2

              def unp(g, cc):
                  sl = pl.ds(g * 16, 16)
                  pk = pcv[b][sl]
                  irv[b][sl] = lax.bitwise_and(pk, 0xFFFF)
                  icv[b][sl] = lax.shift_right_logical(pk, 16)
                  return cc

              lax.fori_loop(0, K2 // 16, unp, 0)
              pltpu.async_copy(hw.at[c].at[icv[b]], av[b], semi[b])
              pltpu.async_copy(wmat.at[c].at[pl.ds(base, K2)], wv[b], semi[b])

      def compute(t, b):
          j = s + NS * t

          @pl.when(j < nchunk)
          def _():
              base = j * K2
              pltpu.make_async_copy(hw.at[c].at[icv[b]], av[b], semi[b]).wait()
              pltpu.make_async_copy(
                  wmat.at[c].at[pl.ds(base, K2)], wv[b], semi[b]).wait()

              j2 = j + 2 * NS

              @pl.when(j2 < nchunk)
              def _():
                  pltpu.async_copy(rc.at[pl.ds(j2 * K2, K2)], pcv[b], semx[b])

              def mrow(i, cc):
                  for rr in range(2):
                      r = i * 2 + rr
                      for kk in range(HALF // 16):
                          sl = pl.ds(kk * 16, 16)
                          av[b][r, sl] = av[b][r, sl] * wv[b][r, sl]
                  return cc

              lax.fori_loop(0, K2 // 2, mrow, 0)
              pltpu.async_copy(av[b], agg_sh.at[irv[b]], semo[b], add=True)

      @pl.when(s < nchunk)
      def _():
          pltpu.sync_copy(rc.at[pl.ds(s * K2, K2)], pcv[0])

      @pl.when(s + NS < nchunk)
      def _():
          pltpu.sync_copy(rc.at[pl.ds((s + NS) * K2, K2)], pcv[1])

      issue(0, 0, True)
      issue(1, 1, True)

      def zstore(i, carry):
          r = i // (HALF // 16)
          kk = (i % (HALF // 16)) * 16
          z_v[r, pl.ds(kk, 16)] = jnp.zeros((16,), jnp.float32)
          return carry

      lax.fori_loop(0, 16 * (HALF // 16), zstore, 0)

      base0 = s * 624
      nrows = jnp.where(s == NS - 1, 640, 624)

      def zcopy(i, carry):
          pltpu.sync_copy(z_v, agg_sh.at[pl.ds(base0 + i * 16, 16)])
          return carry

      lax.fori_loop(0, nrows // 16, zcopy, 0)
      plsc.subcore_barrier()

      compute(0, 0)

      def body(tt, carry):
          t0 = 2 * tt
          issue(t0 + 2, 0, False)
          compute(t0 + 1, 1)
          issue(t0 + 3, 1, False)
          compute(t0 + 2, 0)
          return carry

      lax.fori_loop(0, (tm + 1) // 2, body, 0)

      pltpu.make_async_copy(av[0], agg_sh.at[irv[0]], semo[0]).wait()
      pltpu.make_async_copy(av[1], agg_sh.at[irv[1]], semo[1]).wait()

      plsc.subcore_barrier()

      @pl.when(s == NS - 1)
      def _():
          pltpu.sync_copy(agg_sh.at[pl.ds(base0, 640)],
                          agg.at[c].at[pl.ds(base0, 640)])

      @pl.when(s < NS - 1)
      def _():
          pltpu.sync_copy(agg_sh.at[pl.ds(base0, 624)],
                          agg.at[c].at[pl.ds(base0, 624)])

  return pl.kernel(
      _msg_body,
      out_type=jax.ShapeDtypeStruct((2, N_NODES, HALF), jnp.float32),
      mesh=plsc.VectorSubcoreMesh(core_axis_name="c", subcore_axis_name="s"),
      scratch_types=[
          pltpu.VMEM((K2,), jnp.int32),
          pltpu.VMEM((K2,), jnp.int32),
          pltpu.VMEM((K2,), jnp.int32),
          pltpu.VMEM((K2,), jnp.int32),
          pltpu.VMEM((K2,), jnp.int32),
          pltpu.VMEM((K2,), jnp.int32),
          pltpu.VMEM((K2, HALF), jnp.float32),
          pltpu.VMEM((K2, HALF), jnp.float32),
          pltpu.VMEM((K2, HALF), jnp.float32),
          pltpu.VMEM((K2, HALF), jnp.float32),
          pltpu.VMEM((16, HALF), jnp.float32),
          pltpu.VMEM_SHARED((N_NODES, HALF), jnp.float32),
          pltpu.SemaphoreType.DMA,
          pltpu.SemaphoreType.DMA,
          pltpu.SemaphoreType.DMA,
          pltpu.SemaphoreType.DMA,
          pltpu.SemaphoreType.DMA,
          pltpu.SemaphoreType.DMA,
      ],
      compiler_params=pltpu.CompilerParams(needs_layout_passes=False),
  )


_msg_quarter = _make_msg(N_EDGES // (4 * K2))


def _out_body(agga_ref, aggb_ref, aggc_ref, aggd_ref, wout_ref, bout_ref,
              out_ref):
    w = wout_ref[...]
    a0 = (agga_ref[0] + aggb_ref[0]) + (aggc_ref[0] + aggd_ref[0])
    a1 = (agga_ref[1] + aggb_ref[1]) + (aggc_ref[1] + aggd_ref[1])
    acc = jnp.dot(a0, w[:HALF, :], preferred_element_type=jnp.float32)
    acc = acc + jnp.dot(a1, w[HALF:, :], preferred_element_type=jnp.float32)
    out_ref[...] = acc + bout_ref[...][None, :]


def _out_call(agga, aggb, aggc, aggd, Wout, bout):
    BN = 1000
    return pl.pallas_call(
        _out_body,
        grid=(N_NODES // BN,),
        in_specs=[
            pl.BlockSpec((2, BN, HALF), lambda i: (0, i, 0)),
            pl.BlockSpec((2, BN, HALF), lambda i: (0, i, 0)),
            pl.BlockSpec((2, BN, HALF), lambda i: (0, i, 0)),
            pl.BlockSpec((2, BN, HALF), lambda i: (0, i, 0)),
            pl.BlockSpec((N_FILTERS, OUT_CH), lambda i: (0, 0)),
            pl.BlockSpec((OUT_CH,), lambda i: (0,)),
        ],
        out_specs=pl.BlockSpec((BN, OUT_CH), lambda i: (i, 0)),
        out_shape=jax.ShapeDtypeStruct((N_NODES, OUT_CH), jnp.float32),
    )(agga, aggb, aggc, aggd, Wout, bout)


def kernel(h, coord, edge_index, edge_mask, W1, b1, W2, b2, Win, Wout, bout):
    row = edge_index[0].astype(jnp.int32)
    col = edge_index[1].astype(jnp.int32)
    coordf = jnp.concatenate(
        [coord, jnp.zeros((N_NODES, 1), coord.dtype)], axis=1).reshape(-1)

    hw = _hw_call(h, Win)
    d2 = _d2_call(coordf, row, col)
    rc = jnp.bitwise_or(row, jnp.left_shift(col, 16))
    eq = N_EDGES // 4
    aggs = []
    for p in range(4):
        sl = slice(p * eq, (p + 1) * eq)
        wm = _filter_call(d2[sl].reshape(eq, 1), edge_mask[sl],
                          W1, b1, W2, b2)
        aggs.append(_msg_quarter(hw, wm, rc[sl]))
    return _out_call(aggs[0], aggs[1], aggs[2], aggs[3], Wout, bout)
